# Initial kernel scaffold; baseline (speedup 1.0000x reference)
import jax
import jax.numpy as jnp
from jax.experimental import pallas as pl


def kernel(x, edge_index, batch, W1, att_src1, att_dst1, b1, W2, att_src2, att_dst2, b2, lin_W, lin_b):
    raise NotImplementedError("write your pallas kernel here")


# speedup vs baseline: 21.4066x; 21.4066x over previous
import functools

import jax
import jax.numpy as jnp
from jax import lax
from jax.experimental import pallas as pl
from jax.experimental.pallas import tpu as pltpu
from jax.experimental.pallas import tpu_sc as plsc

_N = 10000
_E = 320000
_D = 128
_NC = 2
_NS = 16
_NW = _NC * _NS
_L = 16
_NPAD = 10240
_EP = _E // _NW
_CH = 80
_NCHUNK = _EP // _CH

_ROWS_PER_TILE = _NPAD // _NS


def _lrelu(v):
    return jnp.where(v >= 0, v, 0.2 * v)


def _dense_body(x_ref, w_ref, asv_ref, adv_ref, h_ref, as_ref, ad_ref, gv_ref):
    h = jnp.dot(x_ref[...], w_ref[...], preferred_element_type=jnp.float32)
    h_ref[...] = h
    asr = jnp.sum(h * asv_ref[...], axis=1)
    as_ref[...] = asr
    ad_ref[...] = jnp.sum(h * adv_ref[...], axis=1)
    gv_ref[...] = jnp.full((_L,), jnp.max(asr), jnp.float32)


_dense_call = pl.pallas_call(
    _dense_body,
    out_shape=(
        jax.ShapeDtypeStruct((_N, _D), jnp.float32),
        jax.ShapeDtypeStruct((_N,), jnp.float32),
        jax.ShapeDtypeStruct((_N,), jnp.float32),
        jax.ShapeDtypeStruct((_L,), jnp.float32),
    ),
)


def _combine_dense_body(acc_ref, den_ref, h_ref, as_ref, ad_ref, b_ref,
                        w_ref, asv_ref, adv_ref,
                        h2_ref, as2_ref, ad2_ref, gv2_ref):
    asv = as_ref[...]
    adv = ad_ref[...]
    gmax = jnp.max(asv)
    selfw = jnp.exp(_lrelu(asv + adv) - _lrelu(gmax + adv))
    num = (acc_ref[0, :_N, :] + acc_ref[1, :_N, :]
           + selfw[:, None] * h_ref[...])
    den = den_ref[0, :_N] + den_ref[1, :_N] + selfw + 1e-16
    h1 = jnp.maximum(num / den[:, None] + b_ref[...], 0.0)
    h2 = jnp.dot(h1, w_ref[...], preferred_element_type=jnp.float32)
    h2_ref[...] = h2
    as2r = jnp.sum(h2 * asv_ref[...], axis=1)
    as2_ref[...] = as2r
    ad2_ref[...] = jnp.sum(h2 * adv_ref[...], axis=1)
    gv2_ref[...] = jnp.full((_L,), jnp.max(as2r), jnp.float32)


_combine_dense_call = pl.pallas_call(
    _combine_dense_body,
    out_shape=(
        jax.ShapeDtypeStruct((_N, _D), jnp.float32),
        jax.ShapeDtypeStruct((_N,), jnp.float32),
        jax.ShapeDtypeStruct((_N,), jnp.float32),
        jax.ShapeDtypeStruct((_L,), jnp.float32),
    ),
)


def _final_body(acc_ref, den_ref, h_ref, as_ref, ad_ref, b_ref, batch_ref,
                lw_ref, lb_ref, out_ref, pooled_ref):
    asv = as_ref[...]
    adv = ad_ref[...]
    gmax = jnp.max(asv)
    selfw = jnp.exp(_lrelu(asv + adv) - _lrelu(gmax + adv))
    num = (acc_ref[0, :_N, :] + acc_ref[1, :_N, :]
           + selfw[:, None] * h_ref[...])
    den = den_ref[0, :_N] + den_ref[1, :_N] + selfw + 1e-16
    h2 = jnp.maximum(num / den[:, None] + b_ref[...], 0.0)
    batchf = batch_ref[...].astype(jnp.float32)[:, None]

    def pool_row(b, _):
        row = jnp.max(jnp.where(batchf == b, h2, -jnp.inf), axis=0,
                      keepdims=True)
        pooled_ref[pl.ds(b, 1), :] = row
        return 0

    lax.fori_loop(0, 64, pool_row, 0)
    out_ref[...] = (
        jnp.dot(pooled_ref[...], lw_ref[...],
                preferred_element_type=jnp.float32)
        + lb_ref[...])


_final_call = pl.pallas_call(
    _final_body,
    out_shape=jax.ShapeDtypeStruct((64, 5), jnp.float32),
    scratch_shapes=[pltpu.VMEM((64, _D), jnp.float32)],
)


def _edge_body(src_hbm, dst_hbm, h_hbm, asrc_hbm, adst_hbm, gv_hbm,
               acc_out, den_out,
               asrc_v, adst_v, gv_v, sidx_v, didx_v, w_v, rows_v,
               acc_sh, den_sh, sem):
    cid = lax.axis_index("c")
    sid = lax.axis_index("s")
    wid = sid * _NC + cid

    pltpu.sync_copy(asrc_hbm, asrc_v)
    pltpu.sync_copy(adst_hbm, adst_v)
    pltpu.sync_copy(gv_hbm, gv_v)
    gmax = gv_v[...]

    def zrow(r, _):
        for k in range(_D // _L):
            rows_v[r, pl.ds(k * _L, _L)] = jnp.zeros((_L,), jnp.float32)
        return 0

    lax.fori_loop(0, _CH, zrow, 0)

    def zw(g, _):
        w_v[pl.ds(g * _L, _L)] = jnp.zeros((_L,), jnp.float32)
        return 0

    lax.fori_loop(0, _CH // _L, zw, 0)

    row0 = sid * _ROWS_PER_TILE
    for i in range(_ROWS_PER_TILE // _CH):
        pltpu.sync_copy(rows_v, acc_sh.at[pl.ds(row0 + i * _CH, _CH)])
        pltpu.sync_copy(w_v, den_sh.at[pl.ds(row0 + i * _CH, _CH)])
    plsc.subcore_barrier()

    ebase = wid * _EP

    def chunk(i, _):
        base = ebase + i * _CH
        pltpu.sync_copy(src_hbm.at[pl.ds(base, _CH)], sidx_v)
        pltpu.sync_copy(dst_hbm.at[pl.ds(base, _CH)], didx_v)
        gat = pltpu.async_copy(h_hbm.at[sidx_v], rows_v, sem)
        for g in range(_CH // _L):
            s16 = sidx_v[pl.ds(g * _L, _L)]
            d16 = didx_v[pl.ds(g * _L, _L)]
            a_s = plsc.load_gather(asrc_v, [s16])
            a_d = plsc.load_gather(adst_v, [d16])
            e = _lrelu(a_s + a_d)
            c = _lrelu(gmax + a_d)
            w_v[pl.ds(g * _L, _L)] = jnp.exp(e - c)
        gat.wait()

        def scale(j, _):
            wj = plsc.load_gather(w_v, [jnp.full((_L,), j, jnp.int32)])
            for k in range(_D // _L):
                rows_v[j, pl.ds(k * _L, _L)] = (
                    rows_v[j, pl.ds(k * _L, _L)] * wj)
            return 0

        lax.fori_loop(0, _CH, scale, 0)
        pltpu.sync_copy(w_v, den_sh.at[didx_v], add=True)
        pltpu.sync_copy(rows_v, acc_sh.at[didx_v], add=True)
        return 0

    lax.fori_loop(0, _NCHUNK, chunk, 0)
    plsc.subcore_barrier()

    pltpu.sync_copy(acc_sh.at[pl.ds(row0, _ROWS_PER_TILE)],
                    acc_out.at[cid, pl.ds(row0, _ROWS_PER_TILE)])
    pltpu.sync_copy(den_sh.at[pl.ds(row0, _ROWS_PER_TILE)],
                    den_out.at[cid, pl.ds(row0, _ROWS_PER_TILE)])


@functools.cache
def _build_edge_call():
  return pl.kernel(
    _edge_body,
    out_type=(
        jax.ShapeDtypeStruct((_NC, _NPAD, _D), jnp.float32),
        jax.ShapeDtypeStruct((_NC, _NPAD), jnp.float32),
    ),
    mesh=plsc.VectorSubcoreMesh(
        core_axis_name="c", subcore_axis_name="s",
        num_cores=_NC, num_subcores=_NS),
    compiler_params=pltpu.CompilerParams(needs_layout_passes=False),
    scratch_types=[
        pltpu.VMEM((_N,), jnp.float32),
        pltpu.VMEM((_N,), jnp.float32),
        pltpu.VMEM((_L,), jnp.float32),
        pltpu.VMEM((_CH,), jnp.int32),
        pltpu.VMEM((_CH,), jnp.int32),
        pltpu.VMEM((_CH,), jnp.float32),
        pltpu.VMEM((_CH, _D), jnp.float32),
        pltpu.VMEM_SHARED((_NPAD, _D), jnp.float32),
        pltpu.VMEM_SHARED((_NPAD,), jnp.float32),
        pltpu.SemaphoreType.DMA,
    ],
  )


def kernel(x, edge_index, batch, W1, att_src1, att_dst1, b1,
           W2, att_src2, att_dst2, b2, lin_W, lin_b):
    src = edge_index[0]
    dst = edge_index[1]
    _edge_call = _build_edge_call()
    h1, as1, ad1, gv1 = _dense_call(
        x, W1, att_src1.reshape(1, _D), att_dst1.reshape(1, _D))
    acc1, den1 = _edge_call(src, dst, h1, as1, ad1, gv1)
    h2, as2, ad2, gv2 = _combine_dense_call(
        acc1, den1, h1, as1, ad1, b1.reshape(1, _D), W2,
        att_src2.reshape(1, _D), att_dst2.reshape(1, _D))
    acc2, den2 = _edge_call(src, dst, h2, as2, ad2, gv2)
    return _final_call(acc2, den2, h2, as2, ad2, b2.reshape(1, _D),
                       batch, lin_W, lin_b.reshape(1, 5))


# speedup vs baseline: 33.5233x; 1.5660x over previous
import functools

import jax
import jax.numpy as jnp
from jax import lax
from jax.experimental import pallas as pl
from jax.experimental.pallas import tpu as pltpu
from jax.experimental.pallas import tpu_sc as plsc

_N = 10000
_E = 320000
_D = 128
_NC = 2
_NS = 16
_NW = _NC * _NS
_L = 16
_NPAD = 10240
_EP = _E // _NW
_CH = 96
_NPAIR = 53
_EPAD = 320512
_ACC_STRIPE = _N // _NS


def _lrelu(v):
    return jnp.where(v >= 0, v, 0.2 * v)


def _dense_body(x_ref, w_ref, asv_ref, adv_ref, h_ref, as_ref, ad_ref, gv_ref):
    h = jnp.dot(x_ref[...], w_ref[...], preferred_element_type=jnp.float32)
    h_ref[...] = h
    asr = jnp.sum(h * asv_ref[...], axis=1)
    as_ref[...] = asr
    ad_ref[...] = jnp.sum(h * adv_ref[...], axis=1)
    gv_ref[...] = jnp.full((_L,), jnp.max(asr), jnp.float32)


_dense_call = pl.pallas_call(
    _dense_body,
    out_shape=(
        jax.ShapeDtypeStruct((_N, _D), jnp.float32),
        jax.ShapeDtypeStruct((_N,), jnp.float32),
        jax.ShapeDtypeStruct((_N,), jnp.float32),
        jax.ShapeDtypeStruct((_L,), jnp.float32),
    ),
)


def _combine_dense_body(acc_ref, den_ref, h_ref, as_ref, ad_ref, b_ref,
                        w_ref, asv_ref, adv_ref,
                        h2_ref, as2_ref, ad2_ref, gv2_ref):
    asv = as_ref[...]
    adv = ad_ref[...]
    gmax = jnp.max(asv)
    selfw = jnp.exp(_lrelu(asv + adv) - _lrelu(gmax + adv))
    num = (acc_ref[0, :_N, :] + acc_ref[1, :_N, :]
           + selfw[:, None] * h_ref[...])
    den = den_ref[0, :_N] + den_ref[1, :_N] + selfw + 1e-16
    h1 = jnp.maximum(num / den[:, None] + b_ref[...], 0.0)
    h2 = jnp.dot(h1, w_ref[...], preferred_element_type=jnp.float32)
    h2_ref[...] = h2
    as2r = jnp.sum(h2 * asv_ref[...], axis=1)
    as2_ref[...] = as2r
    ad2_ref[...] = jnp.sum(h2 * adv_ref[...], axis=1)
    gv2_ref[...] = jnp.full((_L,), jnp.max(as2r), jnp.float32)


_combine_dense_call = pl.pallas_call(
    _combine_dense_body,
    out_shape=(
        jax.ShapeDtypeStruct((_N, _D), jnp.float32),
        jax.ShapeDtypeStruct((_N,), jnp.float32),
        jax.ShapeDtypeStruct((_N,), jnp.float32),
        jax.ShapeDtypeStruct((_L,), jnp.float32),
    ),
)


def _final_body(acc_ref, den_ref, h_ref, as_ref, ad_ref, b_ref, batch_ref,
                lw_ref, lb_ref, out_ref, pooled_ref):
    asv = as_ref[...]
    adv = ad_ref[...]
    gmax = jnp.max(asv)
    selfw = jnp.exp(_lrelu(asv + adv) - _lrelu(gmax + adv))
    num = (acc_ref[0, :_N, :] + acc_ref[1, :_N, :]
           + selfw[:, None] * h_ref[...])
    den = den_ref[0, :_N] + den_ref[1, :_N] + selfw + 1e-16
    h2 = jnp.maximum(num / den[:, None] + b_ref[...], 0.0)
    batchf = batch_ref[...].astype(jnp.float32)[:, None]

    def pool_row(b, _):
        row = jnp.max(jnp.where(batchf == b, h2, -jnp.inf), axis=0,
                      keepdims=True)
        pooled_ref[pl.ds(b, 1), :] = row
        return 0

    lax.fori_loop(0, 64, pool_row, 0)
    out_ref[...] = (
        jnp.dot(pooled_ref[...], lw_ref[...],
                preferred_element_type=jnp.float32)
        + lb_ref[...])


_final_call = pl.pallas_call(
    _final_body,
    out_shape=jax.ShapeDtypeStruct((64, 5), jnp.float32),
    scratch_shapes=[pltpu.VMEM((64, _D), jnp.float32)],
)


def _edge_body(src_hbm, dst_hbm, h_hbm, asrc_hbm, adst_hbm, gv_hbm,
               acc_out, den_out,
               asrc_v, adst_v, gv_v,
               sidx0, didx0, w0, rows0, sidx1, didx1, w1, rows1,
               acc_sh, den_sh,
               xsem0, xsem1, gsem0, gsem1, ssem0, ssem1):
    cid = lax.axis_index("c")
    sid = lax.axis_index("s")
    wid = sid * _NC + cid
    ebase = wid * _EP
    sidx = (sidx0, sidx1)
    didx = (didx0, didx1)
    wv = (w0, w1)
    rows = (rows0, rows1)
    xsem = (xsem0, xsem1)
    gsem = (gsem0, gsem1)
    ssem = (ssem0, ssem1)

    pltpu.sync_copy(asrc_hbm, asrc_v)
    pltpu.sync_copy(adst_hbm, adst_v)
    pltpu.sync_copy(gv_hbm, gv_v)
    gmax = gv_v[...]

    def start_idx(c, p):
        base = ebase + c * _CH
        pltpu.async_copy(src_hbm.at[pl.ds(base, _CH)], sidx[p], xsem[p])
        pltpu.async_copy(dst_hbm.at[pl.ds(base, _CH)], didx[p], xsem[p])

    def wait_idx(c, p):
        base = ebase + c * _CH
        pltpu.make_async_copy(
            src_hbm.at[pl.ds(base, _CH)], sidx[p], xsem[p]).wait()
        pltpu.make_async_copy(
            dst_hbm.at[pl.ds(base, _CH)], didx[p], xsem[p]).wait()

    def start_gather(p):
        pltpu.async_copy(h_hbm.at[sidx[p]], rows[p], gsem[p])

    def wait_gather(p):
        pltpu.make_async_copy(h_hbm.at[sidx[p]], rows[p], gsem[p]).wait()

    def start_scatter(p):
        pltpu.async_copy(wv[p], den_sh.at[didx[p]], ssem[p], add=True)
        pltpu.async_copy(rows[p], acc_sh.at[didx[p]], ssem[p], add=True)

    def drain_scatter(p):
        pltpu.make_async_copy(wv[p], den_sh.at[didx[p]], ssem[p]).wait()
        pltpu.make_async_copy(rows[p], acc_sh.at[didx[p]], ssem[p]).wait()

    def compute_w(c, p):
        for g in range(_CH // _L):
            s16 = sidx[p][pl.ds(g * _L, _L)]
            d16 = didx[p][pl.ds(g * _L, _L)]
            a_s = plsc.load_gather(asrc_v, [s16])
            a_d = plsc.load_gather(adst_v, [d16])
            e = _lrelu(a_s + a_d)
            cc = _lrelu(gmax + a_d)
            w16 = jnp.exp(e - cc)
            valid = c * _CH + g * _L < _EP
            wv[p][pl.ds(g * _L, _L)] = jnp.where(
                valid, w16, jnp.zeros((_L,), jnp.float32))

    def scale(p):
        def body(u, _):
            for t in range(4):
                j = 4 * u + t
                wj = plsc.load_gather(wv[p], [jnp.full((_L,), j, jnp.int32)])
                for k in range(_D // _L):
                    rows[p][j, pl.ds(k * _L, _L)] = (
                        rows[p][j, pl.ds(k * _L, _L)] * wj)
            return 0

        lax.fori_loop(0, _CH // 4, body, 0)

    def zrow(r, _):
        for k in range(_D // _L):
            rows0[r, pl.ds(k * _L, _L)] = jnp.zeros((_L,), jnp.float32)
        return 0

    lax.fori_loop(0, _CH, zrow, 0)

    def zw(g, _):
        w0[pl.ds(g * _L, _L)] = jnp.zeros((_L,), jnp.float32)
        return 0

    lax.fori_loop(0, _CH // _L, zw, 0)

    arow0 = sid * _ACC_STRIPE
    off = 0
    for sz in (96, 96, 96, 96, 96, 96, 49):
        pltpu.sync_copy(rows0.at[pl.ds(0, sz)],
                        acc_sh.at[pl.ds(arow0 + off, sz)])
        off += sz
    drow0 = sid * (_NPAD // _NS)
    for i in range(8):
        pltpu.sync_copy(w0.at[pl.ds(0, 80)],
                        den_sh.at[pl.ds(drow0 + i * 80, 80)])
    plsc.subcore_barrier()

    start_idx(0, 0)
    wait_idx(0, 0)
    start_gather(0)

    def pair(i, _):
        a = 2 * i
        b = 2 * i + 1

        @pl.when(i > 0)
        def _():
            drain_scatter(1)

        start_idx(b, 1)
        compute_w(a, 0)
        wait_gather(0)
        scale(0)
        start_scatter(0)
        wait_idx(b, 1)
        start_gather(1)
        compute_w(b, 1)
        drain_scatter(0)
        start_idx(a + 2, 0)
        wait_gather(1)
        scale(1)
        start_scatter(1)
        wait_idx(a + 2, 0)
        start_gather(0)
        return 0

    lax.fori_loop(0, _NPAIR, pair, 0)
    drain_scatter(1)
    wait_gather(0)
    plsc.subcore_barrier()

    pltpu.sync_copy(acc_sh.at[pl.ds(sid * 624, 624)],
                    acc_out.at[cid, pl.ds(sid * 624, 624)])

    @pl.when(sid == 0)
    def _():
        pltpu.sync_copy(acc_sh.at[pl.ds(9984, 16)],
                        acc_out.at[cid, pl.ds(9984, 16)])

    pltpu.sync_copy(den_sh.at[pl.ds(drow0, _NPAD // _NS)],
                    den_out.at[cid, pl.ds(drow0, _NPAD // _NS)])


@functools.cache
def _build_edge_call():
  return pl.kernel(
    _edge_body,
    out_type=(
        jax.ShapeDtypeStruct((_NC, _N, _D), jnp.float32),
        jax.ShapeDtypeStruct((_NC, _NPAD), jnp.float32),
    ),
    mesh=plsc.VectorSubcoreMesh(
        core_axis_name="c", subcore_axis_name="s",
        num_cores=_NC, num_subcores=_NS),
    compiler_params=pltpu.CompilerParams(needs_layout_passes=False),
    scratch_types=[
        pltpu.VMEM((_N,), jnp.float32),
        pltpu.VMEM((_N,), jnp.float32),
        pltpu.VMEM((_L,), jnp.float32),
        pltpu.VMEM((_CH,), jnp.int32),
        pltpu.VMEM((_CH,), jnp.int32),
        pltpu.VMEM((_CH,), jnp.float32),
        pltpu.VMEM((_CH, _D), jnp.float32),
        pltpu.VMEM((_CH,), jnp.int32),
        pltpu.VMEM((_CH,), jnp.int32),
        pltpu.VMEM((_CH,), jnp.float32),
        pltpu.VMEM((_CH, _D), jnp.float32),
        pltpu.VMEM_SHARED((_N, _D), jnp.float32),
        pltpu.VMEM_SHARED((_NPAD,), jnp.float32),
        pltpu.SemaphoreType.DMA,
        pltpu.SemaphoreType.DMA,
        pltpu.SemaphoreType.DMA,
        pltpu.SemaphoreType.DMA,
        pltpu.SemaphoreType.DMA,
        pltpu.SemaphoreType.DMA,
    ],
  )


def kernel(x, edge_index, batch, W1, att_src1, att_dst1, b1,
           W2, att_src2, att_dst2, b2, lin_W, lin_b):
    pad = jnp.zeros((_EPAD - _E,), jnp.int32)
    src = jnp.concatenate([edge_index[0], pad])
    dst = jnp.concatenate([edge_index[1], pad])
    _edge_call = _build_edge_call()
    h1, as1, ad1, gv1 = _dense_call(
        x, W1, att_src1.reshape(1, _D), att_dst1.reshape(1, _D))
    acc1, den1 = _edge_call(src, dst, h1, as1, ad1, gv1)
    h2, as2, ad2, gv2 = _combine_dense_call(
        acc1, den1, h1, as1, ad1, b1.reshape(1, _D), W2,
        att_src2.reshape(1, _D), att_dst2.reshape(1, _D))
    acc2, den2 = _edge_call(src, dst, h2, as2, ad2, gv2)
    return _final_call(acc2, den2, h2, as2, ad2, b2.reshape(1, _D),
                       batch, lin_W, lin_b.reshape(1, 5))


# speedup vs baseline: 38.8575x; 1.1591x over previous
import functools

import jax
import jax.numpy as jnp
from jax import lax
from jax.experimental import pallas as pl
from jax.experimental.pallas import tpu as pltpu
from jax.experimental.pallas import tpu_sc as plsc

_N = 10000
_E = 320000
_D = 128
_NC = 2
_NS = 16
_NW = _NC * _NS
_L = 16
_NPAD = 10240
_EP = _E // _NW
_CH = 96
_NPAIR = 53
_EPAD = 320512
_ACC_STRIPE = _N // _NS


def _lrelu(v):
    return jnp.where(v >= 0, v, 0.2 * v)


def _dense_body(x_ref, w_ref, asv_ref, adv_ref, h_ref, as_ref, ad_ref, gv_ref):
    h = jnp.dot(x_ref[...], w_ref[...], preferred_element_type=jnp.float32)
    h_ref[...] = h
    asr = jnp.sum(h * asv_ref[...], axis=1)
    as_ref[...] = asr
    ad_ref[...] = jnp.sum(h * adv_ref[...], axis=1)
    gv_ref[...] = jnp.full((_L,), jnp.max(asr), jnp.float32)


_dense_call = pl.pallas_call(
    _dense_body,
    out_shape=(
        jax.ShapeDtypeStruct((_N, _D), jnp.float32),
        jax.ShapeDtypeStruct((_N,), jnp.float32),
        jax.ShapeDtypeStruct((_N,), jnp.float32),
        jax.ShapeDtypeStruct((_L,), jnp.float32),
    ),
)


def _combine_dense_body(acc_ref, den_ref, h_ref, as_ref, ad_ref, b_ref,
                        w_ref, asv_ref, adv_ref,
                        h2_ref, as2_ref, ad2_ref, gv2_ref):
    asv = as_ref[...]
    adv = ad_ref[...]
    gmax = jnp.max(asv)
    selfw = jnp.exp(_lrelu(asv + adv) - _lrelu(gmax + adv))
    num = (acc_ref[0, :_N, :] + acc_ref[1, :_N, :]
           + selfw[:, None] * h_ref[...])
    den = den_ref[0, :_N] + den_ref[1, :_N] + selfw + 1e-16
    h1 = jnp.maximum(num / den[:, None] + b_ref[...], 0.0)
    h2 = jnp.dot(h1, w_ref[...], preferred_element_type=jnp.float32)
    h2_ref[...] = h2
    as2r = jnp.sum(h2 * asv_ref[...], axis=1)
    as2_ref[...] = as2r
    ad2_ref[...] = jnp.sum(h2 * adv_ref[...], axis=1)
    gv2_ref[...] = jnp.full((_L,), jnp.max(as2r), jnp.float32)


_combine_dense_call = pl.pallas_call(
    _combine_dense_body,
    out_shape=(
        jax.ShapeDtypeStruct((_N, _D), jnp.float32),
        jax.ShapeDtypeStruct((_N,), jnp.float32),
        jax.ShapeDtypeStruct((_N,), jnp.float32),
        jax.ShapeDtypeStruct((_L,), jnp.float32),
    ),
)


def _final_body(acc_ref, den_ref, h_ref, as_ref, ad_ref, b_ref, batch_ref,
                lw_ref, lb_ref, out_ref, pooled_ref):
    asv = as_ref[...]
    adv = ad_ref[...]
    gmax = jnp.max(asv)
    selfw = jnp.exp(_lrelu(asv + adv) - _lrelu(gmax + adv))
    num = (acc_ref[0, :_N, :] + acc_ref[1, :_N, :]
           + selfw[:, None] * h_ref[...])
    den = den_ref[0, :_N] + den_ref[1, :_N] + selfw + 1e-16
    h2 = jnp.maximum(num / den[:, None] + b_ref[...], 0.0)
    batchf = batch_ref[...].astype(jnp.float32)[:, None]

    def pool_row(b, _):
        row = jnp.max(jnp.where(batchf == b, h2, -jnp.inf), axis=0,
                      keepdims=True)
        pooled_ref[pl.ds(b, 1), :] = row
        return 0

    lax.fori_loop(0, 64, pool_row, 0)
    out_ref[...] = (
        jnp.dot(pooled_ref[...], lw_ref[...],
                preferred_element_type=jnp.float32)
        + lb_ref[...])


_final_call = pl.pallas_call(
    _final_body,
    out_shape=jax.ShapeDtypeStruct((64, 5), jnp.float32),
    scratch_shapes=[pltpu.VMEM((64, _D), jnp.float32)],
)


def _edge_body(src_hbm, dst_hbm, h_hbm, asrc_hbm, adst_hbm, gv_hbm,
               acc_out, den_out,
               asrc_v, adst_v, gv_v,
               sidx0, didx0, w0, rows0, sidx1, didx1, w1, rows1,
               didxs0, didxs1,
               acc_sh, den_sh,
               xsem0, xsem1, gsem0, gsem1, ssem0, ssem1):
    cid = lax.axis_index("c")
    sid = lax.axis_index("s")
    wid = sid * _NC + cid
    ebase = wid * _EP
    sidx = (sidx0, sidx1)
    didx = (didx0, didx1)
    didxs = (didxs0, didxs1)
    wv = (w0, w1)
    rows = (rows0, rows1)
    xsem = (xsem0, xsem1)
    gsem = (gsem0, gsem1)
    ssem = (ssem0, ssem1)

    pltpu.sync_copy(asrc_hbm, asrc_v)
    pltpu.sync_copy(adst_hbm, adst_v)
    pltpu.sync_copy(gv_hbm, gv_v)
    gmax = gv_v[...]

    def start_idx(c, p):
        base = ebase + c * _CH
        pltpu.async_copy(src_hbm.at[pl.ds(base, _CH)], sidx[p], xsem[p])
        pltpu.async_copy(dst_hbm.at[pl.ds(base, _CH)], didx[p], xsem[p])

    def wait_idx(c, p):
        base = ebase + c * _CH
        pltpu.make_async_copy(
            src_hbm.at[pl.ds(base, _CH)], sidx[p], xsem[p]).wait()
        pltpu.make_async_copy(
            dst_hbm.at[pl.ds(base, _CH)], didx[p], xsem[p]).wait()

    def start_gather(p):
        pltpu.async_copy(h_hbm.at[sidx[p]], rows[p], gsem[p])

    def wait_gather(p):
        pltpu.make_async_copy(h_hbm.at[sidx[p]], rows[p], gsem[p]).wait()

    def snap_didx(p):
        for g in range(_CH // _L):
            didxs[p][pl.ds(g * _L, _L)] = didx[p][pl.ds(g * _L, _L)]

    def start_scatter(p):
        pltpu.async_copy(wv[p], den_sh.at[didxs[p]], ssem[p], add=True)
        pltpu.async_copy(rows[p], acc_sh.at[didxs[p]], ssem[p], add=True)

    def drain_scatter(p):
        pltpu.make_async_copy(wv[p], den_sh.at[didxs[p]], ssem[p]).wait()
        pltpu.make_async_copy(rows[p], acc_sh.at[didxs[p]], ssem[p]).wait()

    def compute_w(c, p):
        for g in range(_CH // _L):
            s16 = sidx[p][pl.ds(g * _L, _L)]
            d16 = didx[p][pl.ds(g * _L, _L)]
            a_s = plsc.load_gather(asrc_v, [s16])
            a_d = plsc.load_gather(adst_v, [d16])
            e = _lrelu(a_s + a_d)
            cc = _lrelu(gmax + a_d)
            w16 = jnp.exp(e - cc)
            valid = c * _CH + g * _L < _EP
            wv[p][pl.ds(g * _L, _L)] = jnp.where(
                valid, w16, jnp.zeros((_L,), jnp.float32))

    def scale(p):
        def body(u, _):
            for t in range(4):
                j = 4 * u + t
                wj = plsc.load_gather(wv[p], [jnp.full((_L,), j, jnp.int32)])
                for k in range(_D // _L):
                    rows[p][j, pl.ds(k * _L, _L)] = (
                        rows[p][j, pl.ds(k * _L, _L)] * wj)
            return 0

        lax.fori_loop(0, _CH // 4, body, 0)

    def zrow(r, _):
        for k in range(_D // _L):
            rows0[r, pl.ds(k * _L, _L)] = jnp.zeros((_L,), jnp.float32)
        return 0

    lax.fori_loop(0, _CH, zrow, 0)

    def zw(g, _):
        w0[pl.ds(g * _L, _L)] = jnp.zeros((_L,), jnp.float32)
        return 0

    lax.fori_loop(0, _CH // _L, zw, 0)

    arow0 = sid * _ACC_STRIPE
    off = 0
    for sz in (96, 96, 96, 96, 96, 96, 49):
        pltpu.sync_copy(rows0.at[pl.ds(0, sz)],
                        acc_sh.at[pl.ds(arow0 + off, sz)])
        off += sz
    drow0 = sid * (_NPAD // _NS)
    for i in range(8):
        pltpu.sync_copy(w0.at[pl.ds(0, 80)],
                        den_sh.at[pl.ds(drow0 + i * 80, 80)])
    plsc.subcore_barrier()

    start_idx(0, 0)
    start_idx(1, 1)
    wait_idx(0, 0)
    start_gather(0)

    def pair(i, _):
        a = 2 * i
        b = 2 * i + 1
        compute_w(a, 0)
        snap_didx(0)
        wait_gather(0)
        start_idx(a + 2, 0)

        @pl.when(i > 0)
        def _():
            drain_scatter(1)

        wait_idx(b, 1)
        start_gather(1)
        scale(0)
        start_scatter(0)
        compute_w(b, 1)
        snap_didx(1)
        wait_gather(1)
        start_idx(b + 2, 1)
        scale(1)
        start_scatter(1)
        drain_scatter(0)
        wait_idx(a + 2, 0)
        start_gather(0)
        return 0

    lax.fori_loop(0, _NPAIR, pair, 0)
    drain_scatter(1)
    wait_idx(2 * _NPAIR + 1, 1)
    wait_gather(0)
    plsc.subcore_barrier()

    pltpu.sync_copy(acc_sh.at[pl.ds(sid * 624, 624)],
                    acc_out.at[cid, pl.ds(sid * 624, 624)])

    @pl.when(sid == 0)
    def _():
        pltpu.sync_copy(acc_sh.at[pl.ds(9984, 16)],
                        acc_out.at[cid, pl.ds(9984, 16)])

    pltpu.sync_copy(den_sh.at[pl.ds(drow0, _NPAD // _NS)],
                    den_out.at[cid, pl.ds(drow0, _NPAD // _NS)])


@functools.cache
def _build_edge_call():
  return pl.kernel(
    _edge_body,
    out_type=(
        jax.ShapeDtypeStruct((_NC, _N, _D), jnp.float32),
        jax.ShapeDtypeStruct((_NC, _NPAD), jnp.float32),
    ),
    mesh=plsc.VectorSubcoreMesh(
        core_axis_name="c", subcore_axis_name="s",
        num_cores=_NC, num_subcores=_NS),
    compiler_params=pltpu.CompilerParams(needs_layout_passes=False),
    scratch_types=[
        pltpu.VMEM((_N,), jnp.float32),
        pltpu.VMEM((_N,), jnp.float32),
        pltpu.VMEM((_L,), jnp.float32),
        pltpu.VMEM((_CH,), jnp.int32),
        pltpu.VMEM((_CH,), jnp.int32),
        pltpu.VMEM((_CH,), jnp.float32),
        pltpu.VMEM((_CH, _D), jnp.float32),
        pltpu.VMEM((_CH,), jnp.int32),
        pltpu.VMEM((_CH,), jnp.int32),
        pltpu.VMEM((_CH,), jnp.float32),
        pltpu.VMEM((_CH, _D), jnp.float32),
        pltpu.VMEM((_CH,), jnp.int32),
        pltpu.VMEM((_CH,), jnp.int32),
        pltpu.VMEM_SHARED((_N, _D), jnp.float32),
        pltpu.VMEM_SHARED((_NPAD,), jnp.float32),
        pltpu.SemaphoreType.DMA,
        pltpu.SemaphoreType.DMA,
        pltpu.SemaphoreType.DMA,
        pltpu.SemaphoreType.DMA,
        pltpu.SemaphoreType.DMA,
        pltpu.SemaphoreType.DMA,
    ],
  )


def kernel(x, edge_index, batch, W1, att_src1, att_dst1, b1,
           W2, att_src2, att_dst2, b2, lin_W, lin_b):
    pad = jnp.zeros((_EPAD - _E,), jnp.int32)
    src = jnp.concatenate([edge_index[0], pad])
    dst = jnp.concatenate([edge_index[1], pad])
    _edge_call = _build_edge_call()
    h1, as1, ad1, gv1 = _dense_call(
        x, W1, att_src1.reshape(1, _D), att_dst1.reshape(1, _D))
    acc1, den1 = _edge_call(src, dst, h1, as1, ad1, gv1)
    h2, as2, ad2, gv2 = _combine_dense_call(
        acc1, den1, h1, as1, ad1, b1.reshape(1, _D), W2,
        att_src2.reshape(1, _D), att_dst2.reshape(1, _D))
    acc2, den2 = _edge_call(src, dst, h2, as2, ad2, gv2)
    return _final_call(acc2, den2, h2, as2, ad2, b2.reshape(1, _D),
                       batch, lin_W, lin_b.reshape(1, 5))


# speedup vs baseline: 45.3786x; 1.1678x over previous
import functools

import jax
import jax.numpy as jnp
from jax import lax
from jax.experimental import pallas as pl
from jax.experimental.pallas import tpu as pltpu
from jax.experimental.pallas import tpu_sc as plsc

_N = 10000
_E = 320000
_D = 128
_NC = 2
_NS = 16
_NW = _NC * _NS
_L = 16
_NPAD = 10240
_EP = _E // _NW
_CH = 64
_NTRI = 53
_EPAD = 320512
_ACC_STRIPE = _N // _NS


def _lrelu(v):
    return jnp.where(v >= 0, v, 0.2 * v)


def _dense_body(x_ref, w_ref, asv_ref, adv_ref, h_ref, as_ref, ad_ref, gv_ref):
    h = jnp.dot(x_ref[...], w_ref[...], preferred_element_type=jnp.float32)
    h_ref[...] = h
    asr = jnp.sum(h * asv_ref[...], axis=1)
    as_ref[...] = asr
    ad_ref[...] = jnp.sum(h * adv_ref[...], axis=1)
    gv_ref[...] = jnp.full((_L,), jnp.max(asr), jnp.float32)


_dense_call = pl.pallas_call(
    _dense_body,
    out_shape=(
        jax.ShapeDtypeStruct((_N, _D), jnp.float32),
        jax.ShapeDtypeStruct((_N,), jnp.float32),
        jax.ShapeDtypeStruct((_N,), jnp.float32),
        jax.ShapeDtypeStruct((_L,), jnp.float32),
    ),
)


def _combine_dense_body(acc_ref, den_ref, h_ref, as_ref, ad_ref, b_ref,
                        w_ref, asv_ref, adv_ref,
                        h2_ref, as2_ref, ad2_ref, gv2_ref):
    asv = as_ref[...]
    adv = ad_ref[...]
    gmax = jnp.max(asv)
    selfw = jnp.exp(_lrelu(asv + adv) - _lrelu(gmax + adv))
    num = (acc_ref[0, :_N, :] + acc_ref[1, :_N, :]
           + selfw[:, None] * h_ref[...])
    den = den_ref[0, :_N] + den_ref[1, :_N] + selfw + 1e-16
    h1 = jnp.maximum(num / den[:, None] + b_ref[...], 0.0)
    h2 = jnp.dot(h1, w_ref[...], preferred_element_type=jnp.float32)
    h2_ref[...] = h2
    as2r = jnp.sum(h2 * asv_ref[...], axis=1)
    as2_ref[...] = as2r
    ad2_ref[...] = jnp.sum(h2 * adv_ref[...], axis=1)
    gv2_ref[...] = jnp.full((_L,), jnp.max(as2r), jnp.float32)


_combine_dense_call = pl.pallas_call(
    _combine_dense_body,
    out_shape=(
        jax.ShapeDtypeStruct((_N, _D), jnp.float32),
        jax.ShapeDtypeStruct((_N,), jnp.float32),
        jax.ShapeDtypeStruct((_N,), jnp.float32),
        jax.ShapeDtypeStruct((_L,), jnp.float32),
    ),
)


def _final_body(acc_ref, den_ref, h_ref, as_ref, ad_ref, b_ref, batch_ref,
                lw_ref, lb_ref, out_ref, pooled_ref):
    asv = as_ref[...]
    adv = ad_ref[...]
    gmax = jnp.max(asv)
    selfw = jnp.exp(_lrelu(asv + adv) - _lrelu(gmax + adv))
    num = (acc_ref[0, :_N, :] + acc_ref[1, :_N, :]
           + selfw[:, None] * h_ref[...])
    den = den_ref[0, :_N] + den_ref[1, :_N] + selfw + 1e-16
    h2 = jnp.maximum(num / den[:, None] + b_ref[...], 0.0)
    batchf = batch_ref[...].astype(jnp.float32)[:, None]

    def pool_row(b, _):
        row = jnp.max(jnp.where(batchf == b, h2, -jnp.inf), axis=0,
                      keepdims=True)
        pooled_ref[pl.ds(b, 1), :] = row
        return 0

    lax.fori_loop(0, 64, pool_row, 0)
    out_ref[...] = (
        jnp.dot(pooled_ref[...], lw_ref[...],
                preferred_element_type=jnp.float32)
        + lb_ref[...])


_final_call = pl.pallas_call(
    _final_body,
    out_shape=jax.ShapeDtypeStruct((64, 5), jnp.float32),
    scratch_shapes=[pltpu.VMEM((64, _D), jnp.float32)],
)


def _edge_body(src_hbm, dst_hbm, h_hbm, asrc_hbm, adst_hbm, gv_hbm,
               acc_out, den_out,
               asrc_v, adst_v, gv_v,
               sidx0, didx0, w0, rows0, sidx1, didx1, w1, rows1,
               sidx2, didx2, w2, rows2, didxs0, didxs1, didxs2,
               acc_sh, den_sh,
               xsem0, xsem1, xsem2, gsem0, gsem1, gsem2,
               ssem0, ssem1, ssem2):
    cid = lax.axis_index("c")
    sid = lax.axis_index("s")
    wid = sid * _NC + cid
    ebase = wid * _EP
    sidx = (sidx0, sidx1, sidx2)
    didx = (didx0, didx1, didx2)
    didxs = (didxs0, didxs1, didxs2)
    wv = (w0, w1, w2)
    rows = (rows0, rows1, rows2)
    xsem = (xsem0, xsem1, xsem2)
    gsem = (gsem0, gsem1, gsem2)
    ssem = (ssem0, ssem1, ssem2)

    pltpu.sync_copy(asrc_hbm, asrc_v)
    pltpu.sync_copy(adst_hbm, adst_v)
    pltpu.sync_copy(gv_hbm, gv_v)
    gmax = gv_v[...]

    def start_idx(c, p):
        base = ebase + c * _CH
        pltpu.async_copy(src_hbm.at[pl.ds(base, _CH)], sidx[p], xsem[p])
        pltpu.async_copy(dst_hbm.at[pl.ds(base, _CH)], didx[p], xsem[p])

    def wait_idx(c, p):
        base = ebase + c * _CH
        pltpu.make_async_copy(
            src_hbm.at[pl.ds(base, _CH)], sidx[p], xsem[p]).wait()
        pltpu.make_async_copy(
            dst_hbm.at[pl.ds(base, _CH)], didx[p], xsem[p]).wait()

    def start_gather(p):
        pltpu.async_copy(h_hbm.at[sidx[p]], rows[p], gsem[p])

    def wait_gather(p):
        pltpu.make_async_copy(h_hbm.at[sidx[p]], rows[p], gsem[p]).wait()

    def snap_didx(p):
        for g in range(_CH // _L):
            didxs[p][pl.ds(g * _L, _L)] = didx[p][pl.ds(g * _L, _L)]

    def start_scatter(p):
        pltpu.async_copy(wv[p], den_sh.at[didxs[p]], ssem[p], add=True)
        pltpu.async_copy(rows[p], acc_sh.at[didxs[p]], ssem[p], add=True)

    def drain_scatter(p):
        pltpu.make_async_copy(wv[p], den_sh.at[didxs[p]], ssem[p]).wait()
        pltpu.make_async_copy(rows[p], acc_sh.at[didxs[p]], ssem[p]).wait()

    def compute_w(c, p):
        for g in range(_CH // _L):
            s16 = sidx[p][pl.ds(g * _L, _L)]
            d16 = didx[p][pl.ds(g * _L, _L)]
            a_s = plsc.load_gather(asrc_v, [s16])
            a_d = plsc.load_gather(adst_v, [d16])
            e = _lrelu(a_s + a_d)
            cc = _lrelu(gmax + a_d)
            w16 = jnp.exp(e - cc)
            valid = c * _CH + g * _L < _EP
            wv[p][pl.ds(g * _L, _L)] = jnp.where(
                valid, w16, jnp.zeros((_L,), jnp.float32))

    def scale(p):
        def body(u, _):
            for t in range(4):
                j = 4 * u + t
                wj = plsc.load_gather(wv[p], [jnp.full((_L,), j, jnp.int32)])
                for k in range(_D // _L):
                    rows[p][j, pl.ds(k * _L, _L)] = (
                        rows[p][j, pl.ds(k * _L, _L)] * wj)
            return 0

        lax.fori_loop(0, _CH // 4, body, 0)

    def zrow(r, _):
        for k in range(_D // _L):
            rows0[r, pl.ds(k * _L, _L)] = jnp.zeros((_L,), jnp.float32)
        return 0

    lax.fori_loop(0, _CH, zrow, 0)

    def zw(g, _):
        w0[pl.ds(g * _L, _L)] = jnp.zeros((_L,), jnp.float32)
        return 0

    lax.fori_loop(0, _CH // _L, zw, 0)

    arow0 = sid * _ACC_STRIPE
    off = 0
    for sz in (64, 64, 64, 64, 64, 64, 64, 64, 64, 49):
        pltpu.sync_copy(rows0.at[pl.ds(0, sz)],
                        acc_sh.at[pl.ds(arow0 + off, sz)])
        off += sz
    drow0 = sid * (_NPAD // _NS)
    for i in range(10):
        pltpu.sync_copy(w0, den_sh.at[pl.ds(drow0 + i * 64, 64)])
    plsc.subcore_barrier()

    def work(ch, p):
        compute_w(ch, p)
        snap_didx(p)
        wait_gather(p)
        start_idx(ch + 3, p)
        scale(p)
        start_scatter(p)

    def turnover(p, ch, do_drain):
        @pl.when(do_drain)
        def _():
            drain_scatter(p)

        wait_idx(ch, p)
        start_gather(p)

    start_idx(0, 0)
    start_idx(1, 1)
    start_idx(2, 2)
    wait_idx(0, 0)
    start_gather(0)

    def triple(i, _):
        a = 3 * i
        turnover(1, a + 1, i > 0)
        work(a, 0)
        turnover(2, a + 2, i > 0)
        work(a + 1, 1)
        turnover(0, a + 3, i >= 0)
        work(a + 2, 2)
        return 0

    lax.fori_loop(0, _NTRI, triple, 0)
    last = 3 * _NTRI - 1
    drain_scatter(1)
    drain_scatter(2)
    wait_gather(0)
    wait_idx(last + 2, 1)
    wait_idx(last + 3, 2)
    plsc.subcore_barrier()

    pltpu.sync_copy(acc_sh.at[pl.ds(sid * 624, 624)],
                    acc_out.at[cid, pl.ds(sid * 624, 624)])

    @pl.when(sid == 0)
    def _():
        pltpu.sync_copy(acc_sh.at[pl.ds(9984, 16)],
                        acc_out.at[cid, pl.ds(9984, 16)])

    pltpu.sync_copy(den_sh.at[pl.ds(drow0, _NPAD // _NS)],
                    den_out.at[cid, pl.ds(drow0, _NPAD // _NS)])


@functools.cache
def _build_edge_call():
  return pl.kernel(
    _edge_body,
    out_type=(
        jax.ShapeDtypeStruct((_NC, _N, _D), jnp.float32),
        jax.ShapeDtypeStruct((_NC, _NPAD), jnp.float32),
    ),
    mesh=plsc.VectorSubcoreMesh(
        core_axis_name="c", subcore_axis_name="s",
        num_cores=_NC, num_subcores=_NS),
    compiler_params=pltpu.CompilerParams(needs_layout_passes=False),
    scratch_types=[
        pltpu.VMEM((_N,), jnp.float32),
        pltpu.VMEM((_N,), jnp.float32),
        pltpu.VMEM((_L,), jnp.float32),
        pltpu.VMEM((_CH,), jnp.int32),
        pltpu.VMEM((_CH,), jnp.int32),
        pltpu.VMEM((_CH,), jnp.float32),
        pltpu.VMEM((_CH, _D), jnp.float32),
        pltpu.VMEM((_CH,), jnp.int32),
        pltpu.VMEM((_CH,), jnp.int32),
        pltpu.VMEM((_CH,), jnp.float32),
        pltpu.VMEM((_CH, _D), jnp.float32),
        pltpu.VMEM((_CH,), jnp.int32),
        pltpu.VMEM((_CH,), jnp.int32),
        pltpu.VMEM((_CH,), jnp.float32),
        pltpu.VMEM((_CH, _D), jnp.float32),
        pltpu.VMEM((_CH,), jnp.int32),
        pltpu.VMEM((_CH,), jnp.int32),
        pltpu.VMEM((_CH,), jnp.int32),
        pltpu.VMEM_SHARED((_N, _D), jnp.float32),
        pltpu.VMEM_SHARED((_NPAD,), jnp.float32),
        pltpu.SemaphoreType.DMA,
        pltpu.SemaphoreType.DMA,
        pltpu.SemaphoreType.DMA,
        pltpu.SemaphoreType.DMA,
        pltpu.SemaphoreType.DMA,
        pltpu.SemaphoreType.DMA,
        pltpu.SemaphoreType.DMA,
        pltpu.SemaphoreType.DMA,
        pltpu.SemaphoreType.DMA,
    ],
  )


def kernel(x, edge_index, batch, W1, att_src1, att_dst1, b1,
           W2, att_src2, att_dst2, b2, lin_W, lin_b):
    pad = jnp.zeros((_EPAD - _E,), jnp.int32)
    src = jnp.concatenate([edge_index[0], pad])
    dst = jnp.concatenate([edge_index[1], pad])
    _edge_call = _build_edge_call()
    h1, as1, ad1, gv1 = _dense_call(
        x, W1, att_src1.reshape(1, _D), att_dst1.reshape(1, _D))
    acc1, den1 = _edge_call(src, dst, h1, as1, ad1, gv1)
    h2, as2, ad2, gv2 = _combine_dense_call(
        acc1, den1, h1, as1, ad1, b1.reshape(1, _D), W2,
        att_src2.reshape(1, _D), att_dst2.reshape(1, _D))
    acc2, den2 = _edge_call(src, dst, h2, as2, ad2, gv2)
    return _final_call(acc2, den2, h2, as2, ad2, b2.reshape(1, _D),
                       batch, lin_W, lin_b.reshape(1, 5))


# speedup vs baseline: 46.2899x; 1.0201x over previous
import functools

import jax
import jax.numpy as jnp
from jax import lax
from jax.experimental import pallas as pl
from jax.experimental.pallas import tpu as pltpu
from jax.experimental.pallas import tpu_sc as plsc

_N = 10000
_E = 320000
_D = 128
_NC = 2
_NS = 16
_NW = _NC * _NS
_L = 16
_NPAD = 10240
_EP = _E // _NW
_CH = 64
_NTRI = 53
_EPAD = 320512
_ACC_STRIPE = _N // _NS


def _lrelu(v):
    return jnp.where(v >= 0, v, 0.2 * v)


def _dense_body(x_ref, w_ref, asv_ref, adv_ref, h_ref, as_ref, ad_ref, gv_ref):
    h = jnp.dot(x_ref[...], w_ref[...], preferred_element_type=jnp.float32)
    h_ref[...] = h
    asr = jnp.sum(h * asv_ref[...], axis=1)
    as_ref[...] = asr
    ad_ref[...] = jnp.sum(h * adv_ref[...], axis=1)
    gv_ref[...] = jnp.full((_L,), jnp.max(asr), jnp.float32)


_dense_call = pl.pallas_call(
    _dense_body,
    out_shape=(
        jax.ShapeDtypeStruct((_N, _D), jnp.float32),
        jax.ShapeDtypeStruct((_N,), jnp.float32),
        jax.ShapeDtypeStruct((_N,), jnp.float32),
        jax.ShapeDtypeStruct((_L,), jnp.float32),
    ),
)


def _combine_dense_body(acc_ref, den_ref, h_ref, as_ref, ad_ref, b_ref,
                        w_ref, asv_ref, adv_ref,
                        h2_ref, as2_ref, ad2_ref, gv2_ref):
    asv = as_ref[...]
    adv = ad_ref[...]
    gmax = jnp.max(asv)
    selfw = jnp.exp(_lrelu(asv + adv) - _lrelu(gmax + adv))
    num = (acc_ref[0, :_N, :] + acc_ref[1, :_N, :]
           + selfw[:, None] * h_ref[...])
    den = den_ref[0, :_N] + den_ref[1, :_N] + selfw + 1e-16
    h1 = jnp.maximum(num / den[:, None] + b_ref[...], 0.0)
    h2 = jnp.dot(h1, w_ref[...], preferred_element_type=jnp.float32)
    h2_ref[...] = h2
    as2r = jnp.sum(h2 * asv_ref[...], axis=1)
    as2_ref[...] = as2r
    ad2_ref[...] = jnp.sum(h2 * adv_ref[...], axis=1)
    gv2_ref[...] = jnp.full((_L,), jnp.max(as2r), jnp.float32)


_combine_dense_call = pl.pallas_call(
    _combine_dense_body,
    out_shape=(
        jax.ShapeDtypeStruct((_N, _D), jnp.float32),
        jax.ShapeDtypeStruct((_N,), jnp.float32),
        jax.ShapeDtypeStruct((_N,), jnp.float32),
        jax.ShapeDtypeStruct((_L,), jnp.float32),
    ),
)


def _final_body(acc_ref, den_ref, h_ref, as_ref, ad_ref, b_ref, batch_ref,
                lw_ref, lb_ref, out_ref, pooled_ref):
    asv = as_ref[...]
    adv = ad_ref[...]
    gmax = jnp.max(asv)
    selfw = jnp.exp(_lrelu(asv + adv) - _lrelu(gmax + adv))
    num = (acc_ref[0, :_N, :] + acc_ref[1, :_N, :]
           + selfw[:, None] * h_ref[...])
    den = den_ref[0, :_N] + den_ref[1, :_N] + selfw + 1e-16
    h2 = jnp.maximum(num / den[:, None] + b_ref[...], 0.0)
    batchf = batch_ref[...].astype(jnp.float32)[:, None]

    def pool_row(b, _):
        row = jnp.max(jnp.where(batchf == b, h2, -jnp.inf), axis=0,
                      keepdims=True)
        pooled_ref[pl.ds(b, 1), :] = row
        return 0

    lax.fori_loop(0, 64, pool_row, 0)
    out_ref[...] = (
        jnp.dot(pooled_ref[...], lw_ref[...],
                preferred_element_type=jnp.float32)
        + lb_ref[...])


_final_call = pl.pallas_call(
    _final_body,
    out_shape=jax.ShapeDtypeStruct((64, 5), jnp.float32),
    scratch_shapes=[pltpu.VMEM((64, _D), jnp.float32)],
)


def _edge_body(src_hbm, dst_hbm, h_hbm, asrc_hbm, adst_hbm, gv_hbm,
               acc_out, den_out,
               asrc_v, adst_v, gv_v,
               sidx0, didx0, w0, rows0, sidx1, didx1, w1, rows1,
               sidx2, didx2, w2, rows2, didxs0, didxs1, didxs2,
               acc_sh, den_sh,
               xsem0, xsem1, xsem2, gsem0, gsem1, gsem2,
               ssem0, ssem1, ssem2):
    cid = lax.axis_index("c")
    sid = lax.axis_index("s")
    wid = sid * _NC + cid
    ebase = wid * _EP
    sidx = (sidx0, sidx1, sidx2)
    didx = (didx0, didx1, didx2)
    didxs = (didxs0, didxs1, didxs2)
    wv = (w0, w1, w2)
    rows = (rows0, rows1, rows2)
    xsem = (xsem0, xsem1, xsem2)
    gsem = (gsem0, gsem1, gsem2)
    ssem = (ssem0, ssem1, ssem2)

    tab0 = pltpu.async_copy(asrc_hbm, asrc_v, ssem0)
    tab1 = pltpu.async_copy(adst_hbm, adst_v, ssem0)
    tab2 = pltpu.async_copy(gv_hbm, gv_v, ssem0)

    def start_idx(c, p):
        base = ebase + c * _CH
        pltpu.async_copy(src_hbm.at[pl.ds(base, _CH)], sidx[p], xsem[p])
        pltpu.async_copy(dst_hbm.at[pl.ds(base, _CH)], didx[p], xsem[p])

    def wait_idx(c, p):
        base = ebase + c * _CH
        pltpu.make_async_copy(
            src_hbm.at[pl.ds(base, _CH)], sidx[p], xsem[p]).wait()
        pltpu.make_async_copy(
            dst_hbm.at[pl.ds(base, _CH)], didx[p], xsem[p]).wait()

    def start_gather(p):
        pltpu.async_copy(h_hbm.at[sidx[p]], rows[p], gsem[p])

    def wait_gather(p):
        pltpu.make_async_copy(h_hbm.at[sidx[p]], rows[p], gsem[p]).wait()

    def snap_didx(p):
        for g in range(_CH // _L):
            didxs[p][pl.ds(g * _L, _L)] = didx[p][pl.ds(g * _L, _L)]

    def start_scatter(p):
        pltpu.async_copy(wv[p], den_sh.at[didxs[p]], ssem[p], add=True)
        pltpu.async_copy(rows[p], acc_sh.at[didxs[p]], ssem[p], add=True)

    def drain_scatter(p):
        pltpu.make_async_copy(wv[p], den_sh.at[didxs[p]], ssem[p]).wait()
        pltpu.make_async_copy(rows[p], acc_sh.at[didxs[p]], ssem[p]).wait()

    def compute_w(c, p, gmax):
        for g in range(_CH // _L):
            s16 = sidx[p][pl.ds(g * _L, _L)]
            d16 = didx[p][pl.ds(g * _L, _L)]
            a_s = plsc.load_gather(asrc_v, [s16])
            a_d = plsc.load_gather(adst_v, [d16])
            e = _lrelu(a_s + a_d)
            cc = _lrelu(gmax + a_d)
            w16 = jnp.exp(e - cc)
            valid = c * _CH + g * _L < _EP
            wv[p][pl.ds(g * _L, _L)] = jnp.where(
                valid, w16, jnp.zeros((_L,), jnp.float32))

    def scale(p):
        def body(u, _):
            for t in range(4):
                j = 4 * u + t
                wj = plsc.load_gather(wv[p], [jnp.full((_L,), j, jnp.int32)])
                for k in range(_D // _L):
                    rows[p][j, pl.ds(k * _L, _L)] = (
                        rows[p][j, pl.ds(k * _L, _L)] * wj)
            return 0

        lax.fori_loop(0, _CH // 4, body, 0)

    start_idx(0, 0)
    start_idx(1, 1)
    start_idx(2, 2)

    def zrow(r, _):
        for k in range(_D // _L):
            rows0[r, pl.ds(k * _L, _L)] = jnp.zeros((_L,), jnp.float32)
        return 0

    lax.fori_loop(0, _CH, zrow, 0)

    def zw(g, _):
        w0[pl.ds(g * _L, _L)] = jnp.zeros((_L,), jnp.float32)
        return 0

    lax.fori_loop(0, _CH // _L, zw, 0)

    arow0 = sid * _ACC_STRIPE
    off = 0
    for sz in (64, 64, 64, 64, 64, 64, 64, 64, 64, 49):
        pltpu.sync_copy(rows0.at[pl.ds(0, sz)],
                        acc_sh.at[pl.ds(arow0 + off, sz)])
        off += sz
    drow0 = sid * (_NPAD // _NS)
    for i in range(10):
        pltpu.sync_copy(w0, den_sh.at[pl.ds(drow0 + i * 64, 64)])
    tab0.wait()
    tab1.wait()
    tab2.wait()
    gmax = gv_v[...]
    wait_idx(0, 0)
    start_gather(0)
    plsc.subcore_barrier()

    def work(ch, p):
        valid = ch * _CH < _EP

        @pl.when(valid)
        def _():
            compute_w(ch, p, gmax)
            snap_didx(p)

        wait_gather(p)
        start_idx(ch + 3, p)

        @pl.when(valid)
        def _():
            scale(p)
            start_scatter(p)

    def turnover(p, ch, do_drain):
        @pl.when(do_drain & ((ch - 3) * _CH < _EP))
        def _():
            drain_scatter(p)

        wait_idx(ch, p)
        start_gather(p)

    def triple(i, _):
        a = 3 * i
        turnover(1, a + 1, i > 0)
        work(a, 0)
        turnover(2, a + 2, i > 0)
        work(a + 1, 1)
        turnover(0, a + 3, i >= 0)
        work(a + 2, 2)
        return 0

    lax.fori_loop(0, _NTRI, triple, 0)
    last = 3 * _NTRI - 1
    wait_gather(0)
    wait_idx(last + 2, 1)
    wait_idx(last + 3, 2)
    plsc.subcore_barrier()

    pltpu.sync_copy(acc_sh.at[pl.ds(sid * 624, 624)],
                    acc_out.at[cid, pl.ds(sid * 624, 624)])

    @pl.when(sid == 0)
    def _():
        pltpu.sync_copy(acc_sh.at[pl.ds(9984, 16)],
                        acc_out.at[cid, pl.ds(9984, 16)])

    pltpu.sync_copy(den_sh.at[pl.ds(drow0, _NPAD // _NS)],
                    den_out.at[cid, pl.ds(drow0, _NPAD // _NS)])


@functools.cache
def _build_edge_call():
  return pl.kernel(
    _edge_body,
    out_type=(
        jax.ShapeDtypeStruct((_NC, _N, _D), jnp.float32),
        jax.ShapeDtypeStruct((_NC, _NPAD), jnp.float32),
    ),
    mesh=plsc.VectorSubcoreMesh(
        core_axis_name="c", subcore_axis_name="s",
        num_cores=_NC, num_subcores=_NS),
    compiler_params=pltpu.CompilerParams(needs_layout_passes=False),
    scratch_types=[
        pltpu.VMEM((_N,), jnp.float32),
        pltpu.VMEM((_N,), jnp.float32),
        pltpu.VMEM((_L,), jnp.float32),
        pltpu.VMEM((_CH,), jnp.int32),
        pltpu.VMEM((_CH,), jnp.int32),
        pltpu.VMEM((_CH,), jnp.float32),
        pltpu.VMEM((_CH, _D), jnp.float32),
        pltpu.VMEM((_CH,), jnp.int32),
        pltpu.VMEM((_CH,), jnp.int32),
        pltpu.VMEM((_CH,), jnp.float32),
        pltpu.VMEM((_CH, _D), jnp.float32),
        pltpu.VMEM((_CH,), jnp.int32),
        pltpu.VMEM((_CH,), jnp.int32),
        pltpu.VMEM((_CH,), jnp.float32),
        pltpu.VMEM((_CH, _D), jnp.float32),
        pltpu.VMEM((_CH,), jnp.int32),
        pltpu.VMEM((_CH,), jnp.int32),
        pltpu.VMEM((_CH,), jnp.int32),
        pltpu.VMEM_SHARED((_N, _D), jnp.float32),
        pltpu.VMEM_SHARED((_NPAD,), jnp.float32),
        pltpu.SemaphoreType.DMA,
        pltpu.SemaphoreType.DMA,
        pltpu.SemaphoreType.DMA,
        pltpu.SemaphoreType.DMA,
        pltpu.SemaphoreType.DMA,
        pltpu.SemaphoreType.DMA,
        pltpu.SemaphoreType.DMA,
        pltpu.SemaphoreType.DMA,
        pltpu.SemaphoreType.DMA,
    ],
  )


def kernel(x, edge_index, batch, W1, att_src1, att_dst1, b1,
           W2, att_src2, att_dst2, b2, lin_W, lin_b):
    pad = jnp.zeros((_EPAD - _E,), jnp.int32)
    src = jnp.concatenate([edge_index[0], pad])
    dst = jnp.concatenate([edge_index[1], pad])
    _edge_call = _build_edge_call()
    h1, as1, ad1, gv1 = _dense_call(
        x, W1, att_src1.reshape(1, _D), att_dst1.reshape(1, _D))
    acc1, den1 = _edge_call(src, dst, h1, as1, ad1, gv1)
    h2, as2, ad2, gv2 = _combine_dense_call(
        acc1, den1, h1, as1, ad1, b1.reshape(1, _D), W2,
        att_src2.reshape(1, _D), att_dst2.reshape(1, _D))
    acc2, den2 = _edge_call(src, dst, h2, as2, ad2, gv2)
    return _final_call(acc2, den2, h2, as2, ad2, b2.reshape(1, _D),
                       batch, lin_W, lin_b.reshape(1, 5))
